# baseline (speedup 1.0000x reference)
import functools

import jax
import jax.numpy as jnp
from jax import lax
from jax.experimental import pallas as pl
from jax.experimental.pallas import tpu as pltpu
from jax.experimental.pallas import tpu_sc as plsc

ALPHA = 0.7
BETA = 1.5
N = 8192
D = 64
C = 16
W = 16
PPW = N // W
P2 = PPW // 2
NG = PPW // 16
NG2 = P2 // 16


def _sqrt16(s):
  s = jnp.maximum(s, 1e-30)
  i = lax.bitcast_convert_type(s, jnp.int32)
  i = jnp.int32(0x5F3759DF) - lax.shift_right_logical(i, 1)
  y = lax.bitcast_convert_type(i, jnp.float32)
  for _ in range(3):
    y = y * (1.5 - 0.5 * s * y * y)
  return s * y


def _saloss_body(emb_hbm, lab_hbm, pts_hbm, out_hbm,
                 emb_v, lab_v, pts_v, g_v, acc_v, sums_v, mean_v,
                 meanT_v, w_v, buf2_v, red2_v, idx80_v, idx16_v, obuf_v,
                 sh1_v, sh2_v):
  cid = lax.axis_index("c")
  sid = lax.axis_index("s")
  base = sid * PPW
  base2 = base + cid * P2
  loc2 = cid * P2
  iota = lax.iota(jnp.int32, 16)
  zero16 = jnp.zeros((16,), jnp.float32)

  with jax.named_scope("ph_stage_in"):
    pltpu.sync_copy(emb_hbm.at[pl.ds(base * D, PPW * D)], emb_v)
    pltpu.sync_copy(lab_hbm.at[pl.ds(base, PPW)], lab_v)
    pltpu.sync_copy(pts_hbm.at[pl.ds(base2 * 3, P2 * 3)], pts_v)

  @plsc.parallel_loop(0, 80)
  def _zero_acc(r):
    acc_v[r, :] = zero16

  @plsc.parallel_loop(0, 16)
  def _zero_b2(r):
    buf2_v[r, :] = zero16

  @plsc.parallel_loop(0, 5)
  def _fill_idx(r):
    idx80_v[pl.ds(r * 16, 16)] = iota + r * 16
  idx16_v[...] = iota

  @pl.when(sid == 0)
  def _zero_shared():
    pltpu.sync_copy(acc_v, sh1_v)
    pltpu.sync_copy(buf2_v, sh2_v)

  with jax.named_scope("ph_pass1"):
    @plsc.parallel_loop(0, NG)
    def _p1(j):
      lv = lab_v[pl.ds(j * 16, 16)]
      off = j * (16 * D)
      onehot_sum = zero16
      for jj in range(16):
        l = lv[jj]
        for k in range(4):
          v = emb_v[pl.ds(off + jj * D + k * 16, 16)]
          plsc.addupdate(acc_v.at[l * 4 + k], v)
        onehot_sum = onehot_sum + jnp.where(iota == l, 1.0, 0.0)
      plsc.addupdate(acc_v.at[64], onehot_sum)

  iota3 = iota * 3
  with jax.named_scope("ph_g"):
    @plsc.parallel_loop(0, NG2)
    def _g(j):
      b = j * 48
      px = plsc.load_gather(pts_v, [iota3 + b])
      py = plsc.load_gather(pts_v, [iota3 + b + 1])
      pz = plsc.load_gather(pts_v, [iota3 + b + 2])
      nrm = _sqrt16(px * px + py * py + pz * pz)
      g_v[pl.ds(j * 16, 16)] = 1.0 / (1.0 + jnp.exp(-nrm))

  with jax.named_scope("ph_reduce1"):
    plsc.subcore_barrier()
    pltpu.sync_copy(acc_v, sh1_v.at[idx80_v], add=True)
    plsc.subcore_barrier()
    pltpu.sync_copy(sh1_v, sums_v)

  cnt = sums_v[64, :]
  iota_f = iota.astype(jnp.float32)
  Ms = jnp.sum(jnp.where(cnt > 0.0, 1.0, 0.0))
  gate = jnp.logical_and(iota_f >= 1.0, iota_f < Ms)
  w_v[...] = jnp.where(gate, 1.0 / cnt, 0.0)

  for c in range(C):
    cntc = cnt[c]
    for k in range(4):
      r = c * 4 + k
      mean_v[pl.ds(r * 16, 16)] = sums_v[r, :] / cntc

  iota64 = iota * 64
  @plsc.parallel_loop(0, D)
  def _mt(dd):
    meanT_v[dd, :] = plsc.load_gather(mean_v, [iota64 + dd])

  with jax.named_scope("ph_pass2"):
    @plsc.parallel_loop(0, NG2, carry=zero16)
    def _p2(j, intra):
      lv = lab_v[pl.ds(loc2 + j * 16, 16)]
      ebv = iota64 + (loc2 + j * 16) * D

      accs = [zero16, zero16, zero16, zero16]
      for d in range(D):
        ev = plsc.load_gather(emb_v, [ebv + d])
        mv = meanT_v[d, :].at[lv].get(mode="promise_in_bounds")
        diff = ev - mv
        accs[d % 4] = accs[d % 4] + diff * diff
      dsq = (accs[0] + accs[1]) + (accs[2] + accs[3])

      t = jnp.maximum(_sqrt16(dsq) - ALPHA, 0.0)
      val = g_v[pl.ds(j * 16, 16)] * t * t
      wv = plsc.load_gather(w_v, [lv])
      return intra + val * wv
    intra_vec = _p2

  def _irow(d16, dsq):
    mi = mean_v[pl.ds(sid * 64 + d16 * 16, 16)]
    for k in range(16):
      diff = meanT_v[d16 * 16 + k, :] - mi[k]
      dsq = dsq + diff * diff
    return dsq
  dsq2 = lax.fori_loop(0, 4, _irow, zero16)
  hin = jnp.maximum(BETA - _sqrt16(dsq2), 0.0)
  hin = hin * hin
  gate_j = jnp.logical_and(gate, iota != sid)
  widf = lax.convert_element_type(sid, jnp.float32)
  ok_i = jnp.logical_and(jnp.logical_and(sid >= 1, widf < Ms), cid == 0)
  gate_i = jnp.where(ok_i, 1.0, 0.0)
  inter_vec = jnp.where(gate_j, hin, 0.0) * gate_i

  buf2_v[0, :] = intra_vec
  buf2_v[1, :] = inter_vec
  pltpu.sync_copy(buf2_v, sh2_v.at[idx16_v], add=True)
  plsc.subcore_barrier()

  @pl.when(jnp.logical_and(sid == 0, cid == 0))
  def _final0():
    pltpu.sync_copy(sh2_v, red2_v)
    obuf_v[0, :] = red2_v[0, :]
    obuf_v[1, :] = red2_v[1, :]
    obuf_v[2, :] = zero16 + Ms
    pltpu.sync_copy(obuf_v, out_hbm.at[pl.ds(0, 3)])

  @pl.when(jnp.logical_and(sid == 0, cid == 1))
  def _final1():
    pltpu.sync_copy(sh2_v, red2_v)
    obuf_v[0, :] = red2_v[0, :]
    obuf_v[1, :] = red2_v[1, :]
    pltpu.sync_copy(obuf_v.at[pl.ds(0, 2)], out_hbm.at[pl.ds(3, 2)])


@functools.cache
def _build_saloss_sc():
  mesh = plsc.VectorSubcoreMesh(
      core_axis_name="c", subcore_axis_name="s", num_cores=2, num_subcores=16
  )
  return pl.kernel(
      _saloss_body,
      out_type=jax.ShapeDtypeStruct((5, 16), jnp.float32),
      mesh=mesh,
      compiler_params=pltpu.CompilerParams(
          use_tc_tiling_on_sc=False, needs_layout_passes=False,
          disable_bounds_checks=True, disable_semaphore_checks=True,
      ),
      scratch_types=[
          pltpu.VMEM((PPW * D,), jnp.float32),
          pltpu.VMEM((PPW,), jnp.int32),
          pltpu.VMEM((P2 * 3,), jnp.float32),
          pltpu.VMEM((P2,), jnp.float32),
          pltpu.VMEM((80, 16), jnp.float32),
          pltpu.VMEM((80, 16), jnp.float32),
          pltpu.VMEM((C * D,), jnp.float32),
          pltpu.VMEM((D, 16), jnp.float32),
          pltpu.VMEM((16,), jnp.float32),
          pltpu.VMEM((16, 16), jnp.float32),
          pltpu.VMEM((16, 16), jnp.float32),
          pltpu.VMEM((80,), jnp.int32),
          pltpu.VMEM((16,), jnp.int32),
          pltpu.VMEM((3, 16), jnp.float32),
          pltpu.VMEM_SHARED((80, 16), jnp.float32),
          pltpu.VMEM_SHARED((16, 16), jnp.float32),
      ],
  )


def kernel(points, true, embedding):
  emb = embedding.reshape(N * D)
  lab = true.reshape(N)
  pts = points.reshape(N * 3)
  o = _build_saloss_sc()(emb, lab, pts)
  intra = jnp.sum(o[0] + o[3])
  inter = jnp.sum(o[1] + o[4])
  m = o[2, 0]
  res = intra / m + inter / (m * (m - 1.0))
  return res.reshape(1)


# speedup vs baseline: 1.1359x; 1.1359x over previous
import functools

import jax
import jax.numpy as jnp
from jax import lax
from jax.experimental import pallas as pl
from jax.experimental.pallas import tpu as pltpu
from jax.experimental.pallas import tpu_sc as plsc

ALPHA = 0.7
BETA = 1.5
N = 8192
D = 64
C = 16
W = 16
PPW = N // W
P2 = PPW // 2
NG = PPW // 16
NG2 = P2 // 16


def _sqrt16(s):
  s = jnp.maximum(s, 1e-30)
  i = lax.bitcast_convert_type(s, jnp.int32)
  i = jnp.int32(0x5F3759DF) - lax.shift_right_logical(i, 1)
  y = lax.bitcast_convert_type(i, jnp.float32)
  for _ in range(3):
    y = y * (1.5 - 0.5 * s * y * y)
  return s * y


def _saloss_body(emb_hbm, lab_hbm, pts_hbm, out_hbm,
                 emb_v, lab_v, pts_v, g_v, acc_v, sums_v,
                 meanT_v, w_v, buf2_v, red2_v, idx80_v, idx16_v, obuf_v,
                 sh1_v, sh2_v, dma_sem):
  cid = lax.axis_index("c")
  sid = lax.axis_index("s")
  base = sid * PPW
  base2 = base + cid * P2
  loc2 = cid * P2
  iota = lax.iota(jnp.int32, 16)
  zero16 = jnp.zeros((16,), jnp.float32)

  with jax.named_scope("ph_stage_in"):
    emb_cp = pltpu.async_copy(
        emb_hbm.at[pl.ds(base * D, PPW * D)], emb_v, dma_sem)
    pltpu.sync_copy(lab_hbm.at[pl.ds(base, PPW)], lab_v)
    pltpu.sync_copy(pts_hbm.at[pl.ds(base2 * 3, P2 * 3)], pts_v)

  @plsc.parallel_loop(0, 80)
  def _zero_acc(r):
    acc_v[r, :] = zero16

  @plsc.parallel_loop(0, 16)
  def _zero_b2(r):
    buf2_v[r, :] = zero16

  @plsc.parallel_loop(0, 5)
  def _fill_idx(r):
    idx80_v[pl.ds(r * 16, 16)] = iota + r * 16
  idx16_v[...] = iota

  @pl.when(sid == 0)
  def _zero_shared():
    pltpu.sync_copy(acc_v, sh1_v)
    pltpu.sync_copy(buf2_v, sh2_v)

  iota3 = iota * 3
  with jax.named_scope("ph_g"):
    @plsc.parallel_loop(0, NG2)
    def _g(j):
      b = j * 48
      px = plsc.load_gather(pts_v, [iota3 + b])
      py = plsc.load_gather(pts_v, [iota3 + b + 1])
      pz = plsc.load_gather(pts_v, [iota3 + b + 2])
      nrm = _sqrt16(px * px + py * py + pz * pz)
      g_v[pl.ds(j * 16, 16)] = 1.0 / (1.0 + jnp.exp(-nrm))

  emb_cp.wait()

  with jax.named_scope("ph_pass1"):
    @plsc.parallel_loop(0, NG)
    def _p1(j):
      lv = lab_v[pl.ds(j * 16, 16)]
      off = j * (16 * D)
      onehot_sum = zero16
      for jj in range(16):
        l = lv[jj]
        for k in range(4):
          v = emb_v[pl.ds(off + jj * D + k * 16, 16)]
          plsc.addupdate(acc_v.at[l * 4 + k], v)
        onehot_sum = onehot_sum + jnp.where(iota == l, 1.0, 0.0)
      plsc.addupdate(acc_v.at[64], onehot_sum)

  with jax.named_scope("ph_reduce1"):
    plsc.subcore_barrier()
    pltpu.sync_copy(acc_v, sh1_v.at[idx80_v], add=True)
    plsc.subcore_barrier()
    pltpu.sync_copy(sh1_v, sums_v)

  cnt = sums_v[64, :]
  iota_f = iota.astype(jnp.float32)
  Ms = jnp.sum(jnp.where(cnt > 0.0, 1.0, 0.0))
  gate = jnp.logical_and(iota_f >= 1.0, iota_f < Ms)
  w = jnp.where(gate, 1.0 / cnt, 0.0)
  w_v[...] = w

  iota64 = iota * 64
  iota4 = iota * 4
  @plsc.parallel_loop(0, D)
  def _mt(dd):
    rows = iota4 + lax.shift_right_logical(dd, 4)
    cols = jnp.full((16,), dd & 15, jnp.int32)
    meanT_v[dd, :] = plsc.load_gather(sums_v, [rows, cols]) * w

  with jax.named_scope("ph_pass2"):
    @plsc.parallel_loop(0, NG2, carry=zero16)
    def _p2(j, intra):
      lv = lab_v[pl.ds(loc2 + j * 16, 16)]
      ebv = iota64 + (loc2 + j * 16) * D

      def _dim(d8, dsq2):
        a, b = dsq2
        for k in range(8):
          d = d8 * 8 + k
          ev = plsc.load_gather(emb_v, [ebv + d])
          mv = meanT_v[d, :].at[lv].get(mode="promise_in_bounds")
          diff = ev - mv
          if k % 2 == 0:
            a = a + diff * diff
          else:
            b = b + diff * diff
        return (a, b)
      dsqa, dsqb = lax.fori_loop(0, 8, _dim, (zero16, zero16))
      dsq = dsqa + dsqb

      t = jnp.maximum(_sqrt16(dsq) - ALPHA, 0.0)
      val = g_v[pl.ds(j * 16, 16)] * t * t
      wv = plsc.load_gather(w_v, [lv])
      return intra + val * wv
    intra_vec = _p2

  rcs = w.at[jnp.full((16,), sid, jnp.int32)].get(mode="promise_in_bounds")
  def _irow(d16, dsq):
    mi = sums_v[sid * 4 + d16, :] * rcs
    for k in range(16):
      diff = meanT_v[d16 * 16 + k, :] - mi[k]
      dsq = dsq + diff * diff
    return dsq
  dsq2 = lax.fori_loop(0, 4, _irow, zero16)
  hin = jnp.maximum(BETA - _sqrt16(dsq2), 0.0)
  hin = hin * hin
  gate_j = jnp.logical_and(gate, iota != sid)
  widf = lax.convert_element_type(sid, jnp.float32)
  ok_i = jnp.logical_and(jnp.logical_and(sid >= 1, widf < Ms), cid == 0)
  gate_i = jnp.where(ok_i, 1.0, 0.0)
  inter_vec = jnp.where(gate_j, hin, 0.0) * gate_i

  buf2_v[0, :] = intra_vec
  buf2_v[1, :] = inter_vec
  pltpu.sync_copy(buf2_v, sh2_v.at[idx16_v], add=True)
  plsc.subcore_barrier()

  @pl.when(jnp.logical_and(sid == 0, cid == 0))
  def _final0():
    pltpu.sync_copy(sh2_v, red2_v)
    obuf_v[0, :] = red2_v[0, :]
    obuf_v[1, :] = red2_v[1, :]
    obuf_v[2, :] = zero16 + Ms
    pltpu.sync_copy(obuf_v, out_hbm.at[pl.ds(0, 3)])

  @pl.when(jnp.logical_and(sid == 0, cid == 1))
  def _final1():
    pltpu.sync_copy(sh2_v, red2_v)
    obuf_v[0, :] = red2_v[0, :]
    obuf_v[1, :] = red2_v[1, :]
    pltpu.sync_copy(obuf_v.at[pl.ds(0, 2)], out_hbm.at[pl.ds(3, 2)])


@functools.cache
def _build_saloss_sc():
  mesh = plsc.VectorSubcoreMesh(
      core_axis_name="c", subcore_axis_name="s", num_cores=2, num_subcores=16
  )
  return pl.kernel(
      _saloss_body,
      out_type=jax.ShapeDtypeStruct((5, 16), jnp.float32),
      mesh=mesh,
      compiler_params=pltpu.CompilerParams(
          use_tc_tiling_on_sc=False, needs_layout_passes=False,
          disable_bounds_checks=True, disable_semaphore_checks=True,
      ),
      scratch_types=[
          pltpu.VMEM((PPW * D,), jnp.float32),
          pltpu.VMEM((PPW,), jnp.int32),
          pltpu.VMEM((P2 * 3,), jnp.float32),
          pltpu.VMEM((P2,), jnp.float32),
          pltpu.VMEM((80, 16), jnp.float32),
          pltpu.VMEM((80, 16), jnp.float32),
          pltpu.VMEM((D, 16), jnp.float32),
          pltpu.VMEM((16,), jnp.float32),
          pltpu.VMEM((16, 16), jnp.float32),
          pltpu.VMEM((16, 16), jnp.float32),
          pltpu.VMEM((80,), jnp.int32),
          pltpu.VMEM((16,), jnp.int32),
          pltpu.VMEM((3, 16), jnp.float32),
          pltpu.VMEM_SHARED((80, 16), jnp.float32),
          pltpu.VMEM_SHARED((16, 16), jnp.float32),
          pltpu.SemaphoreType.DMA,
      ],
  )


def kernel(points, true, embedding):
  emb = embedding.reshape(N * D)
  lab = true.reshape(N)
  pts = points.reshape(N * 3)
  o = _build_saloss_sc()(emb, lab, pts)
  intra = jnp.sum(o[0] + o[3])
  inter = jnp.sum(o[1] + o[4])
  m = o[2, 0]
  res = intra / m + inter / (m * (m - 1.0))
  return res.reshape(1)
